# baseline (speedup 1.0000x reference)
import functools

import jax
import jax.numpy as jnp
from jax.experimental import pallas as pl
from jax.experimental.pallas import tpu as pltpu
from jax.experimental.pallas import tpu_sc as plsc

NUM_BUCKETS = 32
MAX_DISTANCE = 128
NUM_HEADS = 16
QLEN = 2048
KLEN = 2048
TAB = 4096


def _table_body(off_ref, emb_ref, tab_ref):
    off = off_ref[0]
    n = jax.lax.broadcasted_iota(jnp.int32, (1, 8 * TAB), 1)
    j = (n >> 12) + (n & (TAB - 1))
    rel_pos = j - (QLEN - 1) + off
    n = -rel_pos
    half = NUM_BUCKETS // 2
    ret_hi = jnp.where(n < 0, half, 0)
    na = jnp.abs(n)
    max_exact = half // 2
    is_small = na < max_exact
    eps = jnp.finfo(jnp.float32).eps
    val_if_large = max_exact + (
        jnp.log(na.astype(jnp.float32) / max_exact + eps)
        / jnp.log(MAX_DISTANCE / max_exact)
        * (half - max_exact)
    ).astype(jnp.int32)
    val_if_large = jnp.minimum(val_if_large, half - 1)
    bucket = ret_hi + jnp.where(is_small, na, val_if_large)
    rowid = jax.lax.broadcasted_iota(jnp.int32, (NUM_BUCKETS, 8 * TAB), 0)
    onehot = (bucket == rowid).astype(jnp.float32)
    tab_ref[...] = jnp.dot(
        emb_ref[...], onehot, preferred_element_type=jnp.float32
    )


def _build_table(off, emb):
    flat = pl.pallas_call(
        _table_body,
        out_shape=jax.ShapeDtypeStruct((NUM_HEADS, 8 * TAB), jnp.float32),
        in_specs=[
            pl.BlockSpec(memory_space=pltpu.SMEM),
            pl.BlockSpec(memory_space=pltpu.VMEM),
        ],
        out_specs=pl.BlockSpec(memory_space=pltpu.VMEM),
    )(off, emb)
    return flat


_ROWS = NUM_HEADS * QLEN
_ROWS_PER_TEC = _ROWS // 32
_BATCH = 16


def _sc_expand(table):
    mesh = plsc.VectorSubcoreMesh(core_axis_name="c", subcore_axis_name="s")

    @functools.partial(
        pl.kernel,
        mesh=mesh,
        out_type=jax.ShapeDtypeStruct((_ROWS * KLEN,), jnp.float32),
        scratch_types=[
            pltpu.VMEM((8 * TAB,), jnp.float32),
            pltpu.SemaphoreType.DMA,
        ],
    )
    def body(tab_hbm, out_hbm, tab_v, sem):
        c = jax.lax.axis_index("c")
        s = jax.lax.axis_index("s")
        wid = c * 16 + s
        head = wid // 2
        pltpu.sync_copy(tab_hbm.at[head], tab_v)
        base = wid * _ROWS_PER_TEC

        def chunk(g, carry):
            r0 = base + g * _BATCH
            handles = []
            for b in range(_BATCH):
                r = r0 + b
                q = jax.lax.rem(r, QLEN)
                o = (QLEN - 1) - q
                sh = jax.lax.rem(o, 8)
                o8 = pl.multiple_of(sh * TAB + (o - sh), 8)
                dst = pl.multiple_of(r * KLEN, 8)
                handles.append(
                    pltpu.async_copy(
                        tab_v.at[pl.ds(o8, KLEN)],
                        out_hbm.at[pl.ds(dst, KLEN)],
                        sem,
                    )
                )
            for hd in handles:
                hd.wait()
            return carry

        jax.lax.fori_loop(0, _ROWS_PER_TEC // _BATCH, chunk, 0)

    return body(table)


def kernel(qlen, klen, rel_embedding):
    emb = jnp.asarray(rel_embedding, jnp.float32)
    off = (jnp.asarray(klen, jnp.int32) - jnp.asarray(qlen, jnp.int32)).reshape(1)
    table = _build_table(off, emb)
    out = _sc_expand(table)
    return out.reshape(1, NUM_HEADS, QLEN, KLEN)
